# baseline (speedup 1.0000x reference)
import functools

import jax
import jax.numpy as jnp
from jax import lax
from jax.experimental import pallas as pl
from jax.experimental.pallas import tpu as pltpu
from jax.experimental.pallas import tpu_sc as plsc

VOCAB = 1000000
POS = 512
D = 64
B = 4096
L = 200
N = B * L
NC = 2
NS = 16
NW = NC * NS
T = N // NW
C = 256
SUB = C // 128
K = T // C
NBUF = 3
IDXROWS = N // 128
EPS = 1e-5
BS = 1024


@functools.partial(
    pl.kernel,
    out_type=jax.ShapeDtypeStruct((N // 2, 2 * D), jnp.float32),
    mesh=plsc.VectorSubcoreMesh(core_axis_name="c", subcore_axis_name="s"),
    compiler_params=pltpu.CompilerParams(
        needs_layout_passes=False, use_tc_tiling_on_sc=False),
    scratch_types=(
        [pltpu.VMEM((SUB, 128), jnp.int32) for _ in range(NBUF)]
        + [pltpu.VMEM((SUB, 128), jnp.int32) for _ in range(NBUF)]
        + [pltpu.VMEM((SUB, 128), jnp.int32) for _ in range(NBUF)]
        + [pltpu.VMEM((C, D), jnp.float32) for _ in range(NBUF)]
        + [pltpu.VMEM((C // 2, 2 * D), jnp.float32) for _ in range(NBUF)]
        + [
            pltpu.SemaphoreType.DMA,
            pltpu.SemaphoreType.DMA,
            pltpu.SemaphoreType.DMA,
            pltpu.SemaphoreType.DMA,
        ]
    ),
)
def _gather_sum_kernel(widx_hbm, hidx_hbm, tidx_hbm, ww_hbm, wh_hbm, wt_hbm,
                       out_hbm,
                       wi0, wi1, wi2, hi0, hi1, hi2, ti0, ti1, ti2,
                       rows0, rows1, rows2, pk0, pk1, pk2,
                       isem, wsem, asem, osem):
    wi = [wi0, wi1, wi2]
    hi = [hi0, hi1, hi2]
    ti = [ti0, ti1, ti2]
    rows = [rows0, rows1, rows2]
    pk = [pk0, pk1, pk2]

    wid = lax.axis_index("s") * NC + lax.axis_index("c")
    idx_row0 = wid * (T // 128)
    tok0_w = wid * T

    def fire_idx(k, b):
        row0 = idx_row0 + k * SUB
        pltpu.async_copy(widx_hbm.at[pl.ds(row0, SUB)], wi[b], isem)
        pltpu.async_copy(hidx_hbm.at[pl.ds(row0, SUB)], hi[b], isem)
        pltpu.async_copy(tidx_hbm.at[pl.ds(row0, SUB)], ti[b], isem)

    def wait_idx(b):
        for ref in (wi[b], hi[b], ti[b]):
            pltpu.make_async_copy(widx_hbm.at[pl.ds(idx_row0, SUB)], ref,
                                  isem).wait()

    def fire_word(b):
        for i in range(SUB):
            pltpu.async_copy(ww_hbm.at[wi[b].at[i]],
                             rows[b].at[pl.ds(i * 128, 128)], wsem)

    def wait_word(b):
        for i in range(SUB):
            pltpu.make_async_copy(ww_hbm.at[wi[b].at[i]],
                                  rows[b].at[pl.ds(i * 128, 128)],
                                  wsem).wait()

    def fire_ht(b):
        for i in range(SUB):
            pltpu.async_copy(wh_hbm.at[hi[b].at[i]],
                             rows[b].at[pl.ds(i * 128, 128)], asem,
                             add=True)
            pltpu.async_copy(wt_hbm.at[ti[b].at[i]],
                             rows[b].at[pl.ds(i * 128, 128)], asem,
                             add=True)

    def wait_ht(b):
        for i in range(SUB):
            for _ in range(2):
                pltpu.make_async_copy(wh_hbm.at[hi[b].at[i]],
                                      rows[b].at[pl.ds(i * 128, 128)],
                                      asem).wait()

    def fire_out(k, b):
        row0 = (tok0_w + k * C) // 2
        pltpu.async_copy(pk[b], out_hbm.at[pl.ds(row0, C // 2)], osem)

    def wait_out(b):
        pltpu.make_async_copy(pk[b], out_hbm.at[pl.ds(tok0_w // 2, C // 2)],
                              osem).wait()

    def pack(b):
        for i in range(SUB):
            def grp(g, c2, i=i, b=b):
                for j in range(16):
                    r = i * 128 + g * 16 + j
                    rr = i * 64 + g * 8 + (j >> 1)
                    col = (j & 1) * D
                    for c in range(4):
                        pk[b][rr, pl.ds(col + 16 * c, 16)] = (
                            rows[b][r, pl.ds(16 * c, 16)])
                return c2
            lax.fori_loop(0, 8, grp, 0)

    def iteration(k, p0, p1, p2):
        @pl.when(jnp.logical_and(k >= 1, k + 2 < K))
        def _():
            wait_out(p2)

        @pl.when(k + 2 < K)
        def _():
            fire_idx(k + 2, p2)

        @pl.when(k + 1 < K)
        def _():
            wait_word(p1)

        @pl.when(k + 2 < K)
        def _():
            wait_idx(p2)
            fire_word(p2)
        wait_ht(p0)

        @pl.when(k + 1 < K)
        def _():
            fire_ht(p1)
        pack(p0)
        fire_out(k, p0)

    fire_idx(0, 0)
    wait_idx(0)
    fire_word(0)
    wait_word(0)
    fire_ht(0)
    fire_idx(1, 1)
    wait_idx(1)
    fire_word(1)

    def body(k, carry):
        for p in range(NBUF):
            @pl.when(k % NBUF == p)
            def _(p=p):
                iteration(k, p, (p + 1) % NBUF, (p + 2) % NBUF)
        return carry

    lax.fori_loop(0, K, body, 0)

    for _ in range(NBUF):
        wait_out(0)


def _ln_body(x_ref, g_ref, b_ref, o_ref):
    x = x_ref[...]
    g = g_ref[0, :]
    b = b_ref[0, :]

    def norm(v):
        mean = jnp.mean(v, axis=1, keepdims=True)
        c = v - mean
        var = jnp.mean(c * c, axis=1, keepdims=True)
        return c * lax.rsqrt(var + EPS) * g + b

    y = jnp.stack([norm(x[:, :D]), norm(x[:, D:])], axis=1)
    o_ref[...] = y.reshape(2 * BS, D)


_ln_kernel = pl.pallas_call(
    _ln_body,
    grid=(N // 2 // BS,),
    in_specs=[
        pl.BlockSpec((BS, 2 * D), lambda i: (i, 0)),
        pl.BlockSpec((1, D), lambda i: (0, 0)),
        pl.BlockSpec((1, D), lambda i: (0, 0)),
    ],
    out_specs=pl.BlockSpec((2 * BS, D), lambda i: (i, 0)),
    out_shape=jax.ShapeDtypeStruct((N, D), jnp.float32),
)


def kernel(word, head, tail, W_word, W_head, W_tail, gamma, beta):
    wf = word.reshape(IDXROWS, 128)
    hf = head.reshape(IDXROWS, 128)
    tf = tail.reshape(IDXROWS, 128)
    sums = _gather_sum_kernel(wf, hf, tf, W_word, W_head, W_tail)
    out = _ln_kernel(sums, gamma.reshape(1, D), beta.reshape(1, D))
    return out.reshape(B, L, D)


# speedup vs baseline: 1.3098x; 1.3098x over previous
import functools

import jax
import jax.numpy as jnp
from jax import lax
from jax.experimental import pallas as pl
from jax.experimental.pallas import tpu as pltpu
from jax.experimental.pallas import tpu_sc as plsc

VOCAB = 1000000
POS = 512
D = 64
B = 4096
L = 200
N = B * L
NC = 2
NS = 16
NW = NC * NS
T = N // NW
C = 256
SUB = C // 128
K = T // C
NBUF = 3
IDXROWS = N // 128
EPS = 1e-5


@functools.partial(
    pl.kernel,
    out_type=jax.ShapeDtypeStruct((N, D), jnp.float32),
    mesh=plsc.VectorSubcoreMesh(core_axis_name="c", subcore_axis_name="s"),
    compiler_params=pltpu.CompilerParams(
        needs_layout_passes=False, use_tc_tiling_on_sc=False),
    scratch_types=(
        [pltpu.VMEM((SUB, 128), jnp.int32) for _ in range(NBUF)]
        + [pltpu.VMEM((SUB, 128), jnp.int32) for _ in range(NBUF)]
        + [pltpu.VMEM((SUB, 128), jnp.int32) for _ in range(NBUF)]
        + [pltpu.VMEM((C, D), jnp.float32) for _ in range(NBUF)]
        + [
            pltpu.VMEM((D,), jnp.float32),
            pltpu.VMEM((D,), jnp.float32),
            pltpu.SemaphoreType.DMA,
            pltpu.SemaphoreType.DMA,
            pltpu.SemaphoreType.DMA,
            pltpu.SemaphoreType.DMA,
        ]
    ),
)
def _embed_ln_kernel(widx_hbm, hidx_hbm, tidx_hbm, ww_hbm, wh_hbm, wt_hbm,
                     g_hbm, b_hbm, out_hbm,
                     wi0, wi1, wi2, hi0, hi1, hi2, ti0, ti1, ti2,
                     rows0, rows1, rows2, gv, bv,
                     isem, wsem, asem, osem):
    wi = [wi0, wi1, wi2]
    hi = [hi0, hi1, hi2]
    ti = [ti0, ti1, ti2]
    rows = [rows0, rows1, rows2]

    wid = lax.axis_index("s") * NC + lax.axis_index("c")
    pltpu.sync_copy(g_hbm, gv)
    pltpu.sync_copy(b_hbm, bv)
    gs = [gv[pl.ds(16 * c, 16)] for c in range(4)]
    bs = [bv[pl.ds(16 * c, 16)] for c in range(4)]

    idx_row0 = wid * (T // 128)
    tok0_w = wid * T

    def fire_idx(k, b):
        row0 = idx_row0 + k * SUB
        pltpu.async_copy(widx_hbm.at[pl.ds(row0, SUB)], wi[b], isem)
        pltpu.async_copy(hidx_hbm.at[pl.ds(row0, SUB)], hi[b], isem)
        pltpu.async_copy(tidx_hbm.at[pl.ds(row0, SUB)], ti[b], isem)

    def wait_idx(b):
        for ref in (wi[b], hi[b], ti[b]):
            pltpu.make_async_copy(widx_hbm.at[pl.ds(idx_row0, SUB)], ref,
                                  isem).wait()

    def fire_word(b):
        for i in range(SUB):
            pltpu.async_copy(ww_hbm.at[wi[b].at[i]],
                             rows[b].at[pl.ds(i * 128, 128)], wsem)

    def wait_word(b):
        for i in range(SUB):
            pltpu.make_async_copy(ww_hbm.at[wi[b].at[i]],
                                  rows[b].at[pl.ds(i * 128, 128)],
                                  wsem).wait()

    def fire_ht(b):
        for i in range(SUB):
            pltpu.async_copy(wh_hbm.at[hi[b].at[i]],
                             rows[b].at[pl.ds(i * 128, 128)], asem,
                             add=True)
            pltpu.async_copy(wt_hbm.at[ti[b].at[i]],
                             rows[b].at[pl.ds(i * 128, 128)], asem,
                             add=True)

    def wait_ht(b):
        for i in range(SUB):
            for _ in range(2):
                pltpu.make_async_copy(wh_hbm.at[hi[b].at[i]],
                                      rows[b].at[pl.ds(i * 128, 128)],
                                      asem).wait()

    def fire_out(k, b):
        tok0 = tok0_w + k * C
        pltpu.async_copy(rows[b], out_hbm.at[pl.ds(tok0, C)], osem)

    def wait_out(b):
        pltpu.make_async_copy(rows[b], out_hbm.at[pl.ds(tok0_w, C)],
                              osem).wait()

    def compute(b):
        for i in range(SUB):
            def grp(g, c2, i=i, b=b):
                for j in range(16):
                    r = i * 128 + g * 16 + j
                    xs = [rows[b][r, pl.ds(16 * c, 16)] for c in range(4)]
                    s = (xs[0] + xs[1]) + (xs[2] + xs[3])
                    q = (xs[0] * xs[0] + xs[1] * xs[1]
                         + xs[2] * xs[2] + xs[3] * xs[3])
                    mean = jnp.broadcast_to(jnp.sum(s) * (1.0 / D), (16,))
                    msq = jnp.broadcast_to(jnp.sum(q) * (1.0 / D), (16,))
                    a = msq - mean * mean + EPS
                    bits = lax.bitcast_convert_type(a, jnp.int32)
                    bits = jnp.int32(0x5F3759DF) - (bits >> 1)
                    y = lax.bitcast_convert_type(bits, jnp.float32)
                    for _ in range(3):
                        y = y * (1.5 - 0.5 * a * y * y)
                    for c in range(4):
                        rows[b][r, pl.ds(16 * c, 16)] = (
                            (xs[c] - mean) * y * gs[c] + bs[c])
                return c2
            lax.fori_loop(0, 8, grp, 0)

    def iteration(k, p0, p1, p2):
        @pl.when(jnp.logical_and(k >= 1, k + 2 < K))
        def _():
            wait_out(p2)

        @pl.when(k + 2 < K)
        def _():
            fire_idx(k + 2, p2)

        @pl.when(k + 1 < K)
        def _():
            wait_word(p1)

        @pl.when(k + 2 < K)
        def _():
            wait_idx(p2)
            fire_word(p2)
        wait_ht(p0)

        @pl.when(k + 1 < K)
        def _():
            fire_ht(p1)
        compute(p0)
        fire_out(k, p0)

    fire_idx(0, 0)
    wait_idx(0)
    fire_word(0)
    wait_word(0)
    fire_ht(0)
    fire_idx(1, 1)
    wait_idx(1)
    fire_word(1)

    def body(k, carry):
        for p in range(NBUF):
            @pl.when(k % NBUF == p)
            def _(p=p):
                iteration(k, p, (p + 1) % NBUF, (p + 2) % NBUF)
        return carry

    lax.fori_loop(0, K, body, 0)

    for _ in range(NBUF):
        wait_out(0)


def kernel(word, head, tail, W_word, W_head, W_tail, gamma, beta):
    wf = word.reshape(IDXROWS, 128)
    hf = head.reshape(IDXROWS, 128)
    tf = tail.reshape(IDXROWS, 128)
    out = _embed_ln_kernel(wf, hf, tf, W_word, W_head, W_tail, gamma, beta)
    return out.reshape(B, L, D)


# speedup vs baseline: 1.4806x; 1.1304x over previous
import functools

import jax
import jax.numpy as jnp
from jax import lax
from jax.experimental import pallas as pl
from jax.experimental.pallas import tpu as pltpu
from jax.experimental.pallas import tpu_sc as plsc

VOCAB = 1000000
POS = 512
D = 64
B = 4096
L = 200
N = B * L
NC = 2
NS = 16
NW = NC * NS
T = N // NW
C = 256
SUB = C // 128
K = T // C
NBUF = 3
IDXROWS = N // 128
EPS = 1e-5


@functools.partial(
    pl.kernel,
    out_type=jax.ShapeDtypeStruct((N, D), jnp.float32),
    mesh=plsc.VectorSubcoreMesh(core_axis_name="c", subcore_axis_name="s"),
    compiler_params=pltpu.CompilerParams(
        needs_layout_passes=False, use_tc_tiling_on_sc=False),
    scratch_types=(
        [pltpu.VMEM((SUB, 128), jnp.int32) for _ in range(NBUF)]
        + [pltpu.VMEM((SUB, 128), jnp.int32) for _ in range(NBUF)]
        + [pltpu.VMEM((SUB, 128), jnp.int32) for _ in range(NBUF)]
        + [pltpu.VMEM((C, D), jnp.float32) for _ in range(NBUF)]
        + [
            pltpu.VMEM_SHARED((POS, D), jnp.float32),
            pltpu.VMEM_SHARED((POS, D), jnp.float32),
            pltpu.VMEM((D,), jnp.float32),
            pltpu.VMEM((D,), jnp.float32),
            pltpu.SemaphoreType.DMA,
            pltpu.SemaphoreType.DMA,
            pltpu.SemaphoreType.DMA,
            pltpu.SemaphoreType.DMA,
        ]
    ),
)
def _embed_ln_kernel(widx_hbm, hidx_hbm, tidx_hbm, ww_hbm, wh_hbm, wt_hbm,
                     g_hbm, b_hbm, out_hbm,
                     wi0, wi1, wi2, hi0, hi1, hi2, ti0, ti1, ti2,
                     rows0, rows1, rows2, wh_sh, wt_sh, gv, bv,
                     isem, wsem, asem, osem):
    wi = [wi0, wi1, wi2]
    hi = [hi0, hi1, hi2]
    ti = [ti0, ti1, ti2]
    rows = [rows0, rows1, rows2]

    wid = lax.axis_index("s") * NC + lax.axis_index("c")
    pltpu.sync_copy(g_hbm, gv)
    pltpu.sync_copy(b_hbm, bv)

    @pl.when(lax.axis_index("s") == 0)
    def _():
        pltpu.sync_copy(wh_hbm, wh_sh)
        pltpu.sync_copy(wt_hbm, wt_sh)
    plsc.subcore_barrier()
    gs = [gv[pl.ds(16 * c, 16)] for c in range(4)]
    bs = [bv[pl.ds(16 * c, 16)] for c in range(4)]

    idx_row0 = wid * (T // 128)
    tok0_w = wid * T

    def fire_idx(k, b):
        row0 = idx_row0 + k * SUB
        pltpu.async_copy(widx_hbm.at[pl.ds(row0, SUB)], wi[b], isem)
        pltpu.async_copy(hidx_hbm.at[pl.ds(row0, SUB)], hi[b], isem)
        pltpu.async_copy(tidx_hbm.at[pl.ds(row0, SUB)], ti[b], isem)

    def wait_idx(b):
        for ref in (wi[b], hi[b], ti[b]):
            pltpu.make_async_copy(widx_hbm.at[pl.ds(idx_row0, SUB)], ref,
                                  isem).wait()

    def fire_word(b):
        for i in range(SUB):
            pltpu.async_copy(ww_hbm.at[wi[b].at[i]],
                             rows[b].at[pl.ds(i * 128, 128)], wsem)

    def wait_word(b):
        for i in range(SUB):
            pltpu.make_async_copy(ww_hbm.at[wi[b].at[i]],
                                  rows[b].at[pl.ds(i * 128, 128)],
                                  wsem).wait()

    def fire_ht(b):
        for i in range(SUB):
            pltpu.async_copy(wh_sh.at[hi[b].at[i]],
                             rows[b].at[pl.ds(i * 128, 128)], asem,
                             add=True)
            pltpu.async_copy(wt_sh.at[ti[b].at[i]],
                             rows[b].at[pl.ds(i * 128, 128)], asem,
                             add=True)

    def wait_ht(b):
        for i in range(SUB):
            for _ in range(2):
                pltpu.make_async_copy(wh_sh.at[hi[b].at[i]],
                                      rows[b].at[pl.ds(i * 128, 128)],
                                      asem).wait()

    def fire_out(k, b):
        tok0 = tok0_w + k * C
        pltpu.async_copy(rows[b], out_hbm.at[pl.ds(tok0, C)], osem)

    def wait_out(b):
        pltpu.make_async_copy(rows[b], out_hbm.at[pl.ds(tok0_w, C)],
                              osem).wait()

    def compute(b):
        for i in range(SUB):
            def grp(g, c2, i=i, b=b):
                for j in range(16):
                    r = i * 128 + g * 16 + j
                    xs = [rows[b][r, pl.ds(16 * c, 16)] for c in range(4)]
                    s = (xs[0] + xs[1]) + (xs[2] + xs[3])
                    q = (xs[0] * xs[0] + xs[1] * xs[1]
                         + xs[2] * xs[2] + xs[3] * xs[3])
                    mean = jnp.broadcast_to(jnp.sum(s) * (1.0 / D), (16,))
                    msq = jnp.broadcast_to(jnp.sum(q) * (1.0 / D), (16,))
                    a = msq - mean * mean + EPS
                    bits = lax.bitcast_convert_type(a, jnp.int32)
                    bits = jnp.int32(0x5F3759DF) - (bits >> 1)
                    y = lax.bitcast_convert_type(bits, jnp.float32)
                    for _ in range(3):
                        y = y * (1.5 - 0.5 * a * y * y)
                    for c in range(4):
                        rows[b][r, pl.ds(16 * c, 16)] = (
                            (xs[c] - mean) * y * gs[c] + bs[c])
                return c2
            lax.fori_loop(0, 8, grp, 0)

    def iteration(k, p0, p1, p2):
        @pl.when(jnp.logical_and(k >= 1, k + 2 < K))
        def _():
            wait_out(p2)

        @pl.when(k + 2 < K)
        def _():
            fire_idx(k + 2, p2)

        @pl.when(k + 1 < K)
        def _():
            wait_word(p1)

        @pl.when(k + 2 < K)
        def _():
            wait_idx(p2)
            fire_word(p2)
        wait_ht(p0)

        @pl.when(k + 1 < K)
        def _():
            fire_ht(p1)
        compute(p0)
        fire_out(k, p0)

    fire_idx(0, 0)
    wait_idx(0)
    fire_word(0)
    wait_word(0)
    fire_ht(0)
    fire_idx(1, 1)
    wait_idx(1)
    fire_word(1)

    def body(k, carry):
        for p in range(NBUF):
            @pl.when(k % NBUF == p)
            def _(p=p):
                iteration(k, p, (p + 1) % NBUF, (p + 2) % NBUF)
        return carry

    lax.fori_loop(0, K, body, 0)

    for _ in range(NBUF):
        wait_out(0)


def kernel(word, head, tail, W_word, W_head, W_tail, gamma, beta):
    wf = word.reshape(IDXROWS, 128)
    hf = head.reshape(IDXROWS, 128)
    tf = tail.reshape(IDXROWS, 128)
    out = _embed_ln_kernel(wf, hf, tf, W_word, W_head, W_tail, gamma, beta)
    return out.reshape(B, L, D)
